# Initial kernel scaffold; baseline (speedup 1.0000x reference)
import jax
import jax.numpy as jnp
from jax.experimental import pallas as pl


def kernel(x, edge_index, W1, b1, W2, b2, Wf1, bf1, Wf2, bf2):
    raise NotImplementedError("write your pallas kernel here")


# speedup vs baseline: 13.0797x; 13.0797x over previous
import jax
import jax.numpy as jnp
from jax import lax
from jax.experimental import pallas as pl
from jax.experimental.pallas import tpu as pltpu
from jax.experimental.pallas import tpu_sc as plsc

N = 10000
NPAD = 10240
E = 320000
DF = 128
H = 50
D = 128
FC1 = 512
NCLS = 10

NC = 2
NS = 16
NW = NC * NS
BT = 128
ST = 79
EW = ST * BT
EPAD = NW * EW
RPT = NPAD // NS

_MESH = plsc.VectorSubcoreMesh(core_axis_name="c", subcore_axis_name="s",
                               num_cores=NC, num_subcores=NS)
_Z16 = 16


def _deg_kernel(dst_hbm, out_hbm, acc_sp, dstv, onesb, zb):
    c = lax.axis_index("c")
    s = lax.axis_index("s")
    w = c * NS + s

    for i in range(BT // 16):
        onesb[pl.ds(i * 16, 16)] = jnp.full((16,), 1.0, jnp.float32)

    def zero_z(i, carry):
        zb[pl.ds(i * 16, 16)] = jnp.zeros((16,), jnp.float32)
        return carry

    lax.fori_loop(0, RPT // 16, zero_z, 0)
    pltpu.sync_copy(zb, acc_sp.at[pl.ds(s * RPT, RPT)])

    pltpu.sync_copy(dst_hbm.at[w], dstv)
    plsc.subcore_barrier()

    def body(j, carry):
        pltpu.sync_copy(onesb, acc_sp.at[dstv.at[pl.ds(j * BT, BT)]],
                        add=True)
        return carry

    lax.fori_loop(0, ST, body, 0)
    plsc.subcore_barrier()

    pltpu.sync_copy(acc_sp.at[pl.ds(s * RPT, RPT)],
                    out_hbm.at[c, pl.ds(s * RPT, RPT)])


def _deg_sc(dst2):
    k = pl.kernel(
        _deg_kernel,
        out_type=jax.ShapeDtypeStruct((NC, NPAD), jnp.float32),
        mesh=_MESH,
        scratch_types=[
            pltpu.VMEM_SHARED((NPAD,), jnp.float32),
            pltpu.VMEM((EW,), jnp.int32),
            pltpu.VMEM((BT,), jnp.float32),
            pltpu.VMEM((RPT,), jnp.float32),
        ],
        compiler_params=pltpu.CompilerParams(use_tc_tiling_on_sc=False),
    )
    return k(dst2)


def _scatter_kernel(u_hbm, src_hbm, dst_hbm, out_hbm, acc_sp, srcv, dstv,
                    rows, z16, sem):
    c = lax.axis_index("c")
    s = lax.axis_index("s")
    w = c * NS + s

    for i in range(_Z16):
        for kk in range(D // 16):
            z16[i, pl.ds(kk * 16, 16)] = jnp.zeros((16,), jnp.float32)

    def zero_acc(j, carry):
        pltpu.sync_copy(z16, acc_sp.at[pl.ds(s * RPT + j * _Z16, _Z16)])
        return carry

    lax.fori_loop(0, RPT // _Z16, zero_acc, 0)

    pltpu.sync_copy(src_hbm.at[w], srcv)
    pltpu.sync_copy(dst_hbm.at[w], dstv)
    plsc.subcore_barrier()

    def body(j, carry):
        pltpu.async_copy(u_hbm.at[srcv.at[j]], rows, sem).wait()
        pltpu.sync_copy(rows, acc_sp.at[dstv.at[j]], add=True)
        return carry

    lax.fori_loop(0, ST, body, 0)
    plsc.subcore_barrier()

    pltpu.sync_copy(acc_sp.at[pl.ds(s * RPT, RPT)],
                    out_hbm.at[c, pl.ds(s * RPT, RPT)])


def _scatter_sc(u, src3, dst3):
    k = pl.kernel(
        _scatter_kernel,
        out_type=jax.ShapeDtypeStruct((NC, NPAD, D), jnp.float32),
        mesh=_MESH,
        scratch_types=[
            pltpu.VMEM_SHARED((NPAD, D), jnp.float32),
            pltpu.VMEM((ST, BT), jnp.int32),
            pltpu.VMEM((ST, BT), jnp.int32),
            pltpu.VMEM((BT, D), jnp.float32),
            pltpu.VMEM((_Z16, D), jnp.float32),
            pltpu.SemaphoreType.DMA,
        ],
        compiler_params=pltpu.CompilerParams(use_tc_tiling_on_sc=False),
    )
    return k(u, src3, dst3)


def _u1_body(x_ref, w_ref, dinv_ref, o_ref):
    xw = jnp.dot(x_ref[...], w_ref[...], preferred_element_type=jnp.float32)
    o_ref[...] = dinv_ref[...] * xw


def _u1_tc(xpad, w1p, dinv2):
    grid = NPAD // 1024
    return pl.pallas_call(
        _u1_body,
        grid=(grid,),
        in_specs=[
            pl.BlockSpec((1024, DF), lambda i: (i, 0)),
            pl.BlockSpec((DF, D), lambda i: (0, 0)),
            pl.BlockSpec((1024, 1), lambda i: (i, 0)),
        ],
        out_specs=pl.BlockSpec((1024, D), lambda i: (i, 0)),
        out_shape=jax.ShapeDtypeStruct((NPAD, D), jnp.float32),
    )(xpad, w1p, dinv2)


def _elu(v):
    return jnp.where(v > 0, v, jnp.exp(jnp.minimum(v, 0.0)) - 1.0)


def _layer_body(sa_ref, sb_ref, u_ref, dinv_ref, b_ref, w_ref, o_ref):
    dinv = dinv_ref[...]
    p = dinv * (sa_ref[...] + sb_ref[...] + u_ref[...]) + b_ref[...]
    h = _elu(p)
    o_ref[...] = dinv * jnp.dot(h, w_ref[...],
                                preferred_element_type=jnp.float32)


def _layer_tc(sa, sb, u1, dinv2, b1p, w2p):
    grid = NPAD // 1024
    return pl.pallas_call(
        _layer_body,
        grid=(grid,),
        in_specs=[
            pl.BlockSpec((1024, D), lambda i: (i, 0)),
            pl.BlockSpec((1024, D), lambda i: (i, 0)),
            pl.BlockSpec((1024, D), lambda i: (i, 0)),
            pl.BlockSpec((1024, 1), lambda i: (i, 0)),
            pl.BlockSpec((1, D), lambda i: (0, 0)),
            pl.BlockSpec((D, D), lambda i: (0, 0)),
        ],
        out_specs=pl.BlockSpec((1024, D), lambda i: (i, 0)),
        out_shape=jax.ShapeDtypeStruct((NPAD, D), jnp.float32),
    )(sa, sb, u1, dinv2, b1p, w2p)


def _head_body(sa_ref, sb_ref, u_ref, dinv_ref, b_ref, wf1_ref, bf1_ref,
               wf2_ref, bf2_ref, o_ref, g_acc):
    i = pl.program_id(0)

    @pl.when(i == 0)
    def _():
        g_acc[...] = jnp.zeros_like(g_acc)

    dinv = dinv_ref[...]
    p = dinv * (sa_ref[...] + sb_ref[...] + u_ref[...]) + b_ref[...]
    h = _elu(p)
    rows = i * 1024 + jax.lax.broadcasted_iota(jnp.int32, (1024, D), 0)
    h = jnp.where(rows < N, h, 0.0)
    g_acc[...] += jnp.sum(h, axis=0, keepdims=True)

    @pl.when(i == pl.num_programs(0) - 1)
    def _():
        g = g_acc[...]
        t = jnp.maximum(
            jnp.dot(g, wf1_ref[...], preferred_element_type=jnp.float32)
            + bf1_ref[...], 0.0)
        logits = jnp.dot(t, wf2_ref[...],
                         preferred_element_type=jnp.float32) + bf2_ref[...]
        cols = jax.lax.broadcasted_iota(jnp.int32, (1, 128), 1)
        logits = jnp.where(cols < NCLS, logits, -1e30)
        m = jnp.max(logits, axis=1, keepdims=True)
        e = jnp.exp(logits - m)
        o_ref[...] = e / jnp.sum(e, axis=1, keepdims=True)


def _head_tc(sa, sb, u2, dinv2, b2p, wf1p, bf1p, wf2p, bf2p):
    grid = NPAD // 1024
    return pl.pallas_call(
        _head_body,
        grid=(grid,),
        in_specs=[
            pl.BlockSpec((1024, D), lambda i: (i, 0)),
            pl.BlockSpec((1024, D), lambda i: (i, 0)),
            pl.BlockSpec((1024, D), lambda i: (i, 0)),
            pl.BlockSpec((1024, 1), lambda i: (i, 0)),
            pl.BlockSpec((1, D), lambda i: (0, 0)),
            pl.BlockSpec((D, FC1), lambda i: (0, 0)),
            pl.BlockSpec((1, FC1), lambda i: (0, 0)),
            pl.BlockSpec((FC1, 128), lambda i: (0, 0)),
            pl.BlockSpec((1, 128), lambda i: (0, 0)),
        ],
        out_specs=pl.BlockSpec((1, 128), lambda i: (0, 0)),
        out_shape=jax.ShapeDtypeStruct((1, 128), jnp.float32),
        scratch_shapes=[pltpu.VMEM((1, D), jnp.float32)],
    )(sa, sb, u2, dinv2, b2p, wf1p, bf1p, wf2p, bf2p)


def kernel(x, edge_index, W1, b1, W2, b2, Wf1, bf1, Wf2, bf2):
    npad_e = EPAD - E
    src3 = jnp.concatenate(
        [edge_index[0], jnp.zeros((npad_e,), jnp.int32)]).reshape(NW, ST, BT)
    dst3 = jnp.concatenate(
        [edge_index[1], jnp.full((npad_e,), N, jnp.int32)]).reshape(NW, ST, BT)
    xpad = jnp.pad(x, ((0, NPAD - N), (0, 0)))
    w1p = jnp.pad(W1, ((0, 0), (0, D - H)))
    w2p = jnp.pad(W2, ((0, D - H), (0, D - H)))
    b1p = jnp.pad(b1, (0, D - H)).reshape(1, D)
    b2p = jnp.pad(b2, (0, D - H)).reshape(1, D)
    wf1p = jnp.pad(Wf1, ((0, D - H), (0, 0)))
    bf1p = bf1.reshape(1, FC1)
    wf2p = jnp.pad(Wf2, ((0, 0), (0, 128 - NCLS)))
    bf2p = jnp.pad(bf2, (0, 128 - NCLS)).reshape(1, 128)

    degp = _deg_sc(dst3.reshape(NW, EW))
    deg = degp[0] + degp[1] + 1.0
    dinv2 = lax.rsqrt(deg)[:, None]

    u1 = _u1_tc(xpad, w1p, dinv2)
    s1 = _scatter_sc(u1, src3, dst3)
    u2 = _layer_tc(s1[0], s1[1], u1, dinv2, b1p, w2p)
    s2 = _scatter_sc(u2, src3, dst3)
    outp = _head_tc(s2[0], s2[1], u2, dinv2, b2p, wf1p, bf1p, wf2p, bf2p)
    return outp[0, :NCLS]


# speedup vs baseline: 14.5427x; 1.1118x over previous
import jax
import jax.numpy as jnp
from jax import lax
from jax.experimental import pallas as pl
from jax.experimental.pallas import tpu as pltpu
from jax.experimental.pallas import tpu_sc as plsc

N = 10000
NPAD = 10240
E = 320000
DF = 128
H = 50
D = 128
FC1 = 512
NCLS = 10

NC = 2
NS = 16
NW = NC * NS
BT = 128
ST = 79
EW = ST * BT
EPAD = NW * EW
RPT = NPAD // NS
STH = 40

_MESH = plsc.VectorSubcoreMesh(core_axis_name="c", subcore_axis_name="s",
                               num_cores=NC, num_subcores=NS)
_Z16 = 16


def _deg_kernel(dst_hbm, out_hbm, acc_sp, dstv, onesb, zb):
    c = lax.axis_index("c")
    s = lax.axis_index("s")
    w = c * NS + s

    for i in range(BT // 16):
        onesb[pl.ds(i * 16, 16)] = jnp.full((16,), 1.0, jnp.float32)

    def zero_z(i, carry):
        zb[pl.ds(i * 16, 16)] = jnp.zeros((16,), jnp.float32)
        return carry

    lax.fori_loop(0, RPT // 16, zero_z, 0)
    pltpu.sync_copy(zb, acc_sp.at[pl.ds(s * RPT, RPT)])

    pltpu.sync_copy(dst_hbm.at[w], dstv)
    plsc.subcore_barrier()

    def body(j, carry):
        pltpu.sync_copy(onesb, acc_sp.at[dstv.at[pl.ds(j * BT, BT)]],
                        add=True)
        return carry

    lax.fori_loop(0, ST, body, 0)
    plsc.subcore_barrier()

    pltpu.sync_copy(acc_sp.at[pl.ds(s * RPT, RPT)],
                    out_hbm.at[c, pl.ds(s * RPT, RPT)])


def _deg_sc(dst2):
    k = pl.kernel(
        _deg_kernel,
        out_type=jax.ShapeDtypeStruct((NC, NPAD), jnp.float32),
        mesh=_MESH,
        scratch_types=[
            pltpu.VMEM_SHARED((NPAD,), jnp.float32),
            pltpu.VMEM((EW,), jnp.int32),
            pltpu.VMEM((BT,), jnp.float32),
            pltpu.VMEM((RPT,), jnp.float32),
        ],
        compiler_params=pltpu.CompilerParams(use_tc_tiling_on_sc=False),
    )
    return k(dst2)


def _scatter_kernel(u_hbm, src_hbm, dst_hbm, out_hbm, acc_sp, srcv, dstv,
                    r0, r1, z16, g0, g1, s0, s1, zsem):
    c = lax.axis_index("c")
    s = lax.axis_index("s")
    w = c * NS + s

    for i in range(_Z16):
        for kk in range(D // 16):
            z16[i, pl.ds(kk * 16, 16)] = jnp.zeros((16,), jnp.float32)

    def zero_acc(j, carry):
        pltpu.async_copy(z16, acc_sp.at[pl.ds(s * RPT + j * _Z16, _Z16)],
                         zsem)
        return carry

    lax.fori_loop(0, RPT // _Z16, zero_acc, 0)

    pltpu.make_async_copy(u_hbm.at[pl.ds(0, RPT)],
                          acc_sp.at[pl.ds(s * RPT, RPT)], zsem).wait()
    plsc.subcore_barrier()

    def gather(j, rb, sem):
        pltpu.async_copy(u_hbm.at[srcv.at[j]], rb, sem)

    def scatter(j, rb, sem):
        pltpu.async_copy(rb, acc_sp.at[dstv.at[j]], sem, add=True)

    def wait_rows(sem):
        pltpu.make_async_copy(u_hbm.at[pl.ds(0, BT)], r0, sem).wait()

    def run_phase(base, count):
        pltpu.sync_copy(src_hbm.at[w, pl.ds(base, count)],
                        srcv.at[pl.ds(0, count)])
        pltpu.sync_copy(dst_hbm.at[w, pl.ds(base, count)],
                        dstv.at[pl.ds(0, count)])
        gather(0, r0, g0)
        gather(1, r1, g1)
        odd = count % 2
        iters = (count - 2 - odd) // 2

        def body(jj, carry):
            j = jj * 2
            wait_rows(g0)
            scatter(j, r0, s0)
            wait_rows(g1)
            scatter(j + 1, r1, s1)
            wait_rows(s0)
            gather(j + 2, r0, g0)
            wait_rows(s1)
            gather(j + 3, r1, g1)
            return carry

        lax.fori_loop(0, iters, body, 0)
        wait_rows(g0)
        scatter(2 * iters, r0, s0)
        wait_rows(g1)
        scatter(2 * iters + 1, r1, s1)
        wait_rows(s0)
        if odd:
            gather(count - 1, r0, g0)
            wait_rows(g0)
            scatter(count - 1, r0, s0)
            wait_rows(s0)
        wait_rows(s1)

    run_phase(0, STH)
    run_phase(STH, ST - STH)

    plsc.subcore_barrier()

    pltpu.sync_copy(acc_sp.at[pl.ds(s * RPT, RPT)],
                    out_hbm.at[c, pl.ds(s * RPT, RPT)])


def _scatter_sc(u, src3, dst3):
    k = pl.kernel(
        _scatter_kernel,
        out_type=jax.ShapeDtypeStruct((NC, NPAD, D), jnp.float32),
        mesh=_MESH,
        scratch_types=[
            pltpu.VMEM_SHARED((NPAD, D), jnp.float32),
            pltpu.VMEM((STH, BT), jnp.int32),
            pltpu.VMEM((STH, BT), jnp.int32),
            pltpu.VMEM((BT, D), jnp.float32),
            pltpu.VMEM((BT, D), jnp.float32),
            pltpu.VMEM((_Z16, D), jnp.float32),
            pltpu.SemaphoreType.DMA,
            pltpu.SemaphoreType.DMA,
            pltpu.SemaphoreType.DMA,
            pltpu.SemaphoreType.DMA,
            pltpu.SemaphoreType.DMA,
        ],
        compiler_params=pltpu.CompilerParams(use_tc_tiling_on_sc=False),
    )
    return k(u, src3, dst3)


def _u1_body(x_ref, w_ref, dinv_ref, o_ref):
    xw = jnp.dot(x_ref[...], w_ref[...], preferred_element_type=jnp.float32)
    o_ref[...] = dinv_ref[...] * xw


def _u1_tc(xpad, w1p, dinv2):
    grid = NPAD // 1024
    return pl.pallas_call(
        _u1_body,
        grid=(grid,),
        in_specs=[
            pl.BlockSpec((1024, DF), lambda i: (i, 0)),
            pl.BlockSpec((DF, D), lambda i: (0, 0)),
            pl.BlockSpec((1024, 1), lambda i: (i, 0)),
        ],
        out_specs=pl.BlockSpec((1024, D), lambda i: (i, 0)),
        out_shape=jax.ShapeDtypeStruct((NPAD, D), jnp.float32),
    )(xpad, w1p, dinv2)


def _elu(v):
    return jnp.where(v > 0, v, jnp.exp(jnp.minimum(v, 0.0)) - 1.0)


def _layer_body(sa_ref, sb_ref, u_ref, dinv_ref, b_ref, w_ref, o_ref):
    dinv = dinv_ref[...]
    p = dinv * (sa_ref[...] + sb_ref[...] + u_ref[...]) + b_ref[...]
    h = _elu(p)
    o_ref[...] = dinv * jnp.dot(h, w_ref[...],
                                preferred_element_type=jnp.float32)


def _layer_tc(sa, sb, u1, dinv2, b1p, w2p):
    grid = NPAD // 1024
    return pl.pallas_call(
        _layer_body,
        grid=(grid,),
        in_specs=[
            pl.BlockSpec((1024, D), lambda i: (i, 0)),
            pl.BlockSpec((1024, D), lambda i: (i, 0)),
            pl.BlockSpec((1024, D), lambda i: (i, 0)),
            pl.BlockSpec((1024, 1), lambda i: (i, 0)),
            pl.BlockSpec((1, D), lambda i: (0, 0)),
            pl.BlockSpec((D, D), lambda i: (0, 0)),
        ],
        out_specs=pl.BlockSpec((1024, D), lambda i: (i, 0)),
        out_shape=jax.ShapeDtypeStruct((NPAD, D), jnp.float32),
    )(sa, sb, u1, dinv2, b1p, w2p)


def _head_body(sa_ref, sb_ref, u_ref, dinv_ref, b_ref, wf1_ref, bf1_ref,
               wf2_ref, bf2_ref, o_ref, g_acc):
    i = pl.program_id(0)

    @pl.when(i == 0)
    def _():
        g_acc[...] = jnp.zeros_like(g_acc)

    dinv = dinv_ref[...]
    p = dinv * (sa_ref[...] + sb_ref[...] + u_ref[...]) + b_ref[...]
    h = _elu(p)
    rows = i * 1024 + jax.lax.broadcasted_iota(jnp.int32, (1024, D), 0)
    h = jnp.where(rows < N, h, 0.0)
    g_acc[...] += jnp.sum(h, axis=0, keepdims=True)

    @pl.when(i == pl.num_programs(0) - 1)
    def _():
        g = g_acc[...]
        t = jnp.maximum(
            jnp.dot(g, wf1_ref[...], preferred_element_type=jnp.float32)
            + bf1_ref[...], 0.0)
        logits = jnp.dot(t, wf2_ref[...],
                         preferred_element_type=jnp.float32) + bf2_ref[...]
        cols = jax.lax.broadcasted_iota(jnp.int32, (1, 128), 1)
        logits = jnp.where(cols < NCLS, logits, -1e30)
        m = jnp.max(logits, axis=1, keepdims=True)
        e = jnp.exp(logits - m)
        o_ref[...] = e / jnp.sum(e, axis=1, keepdims=True)


def _head_tc(sa, sb, u2, dinv2, b2p, wf1p, bf1p, wf2p, bf2p):
    grid = NPAD // 1024
    return pl.pallas_call(
        _head_body,
        grid=(grid,),
        in_specs=[
            pl.BlockSpec((1024, D), lambda i: (i, 0)),
            pl.BlockSpec((1024, D), lambda i: (i, 0)),
            pl.BlockSpec((1024, D), lambda i: (i, 0)),
            pl.BlockSpec((1024, 1), lambda i: (i, 0)),
            pl.BlockSpec((1, D), lambda i: (0, 0)),
            pl.BlockSpec((D, FC1), lambda i: (0, 0)),
            pl.BlockSpec((1, FC1), lambda i: (0, 0)),
            pl.BlockSpec((FC1, 128), lambda i: (0, 0)),
            pl.BlockSpec((1, 128), lambda i: (0, 0)),
        ],
        out_specs=pl.BlockSpec((1, 128), lambda i: (0, 0)),
        out_shape=jax.ShapeDtypeStruct((1, 128), jnp.float32),
        scratch_shapes=[pltpu.VMEM((1, D), jnp.float32)],
    )(sa, sb, u2, dinv2, b2p, wf1p, bf1p, wf2p, bf2p)


def kernel(x, edge_index, W1, b1, W2, b2, Wf1, bf1, Wf2, bf2):
    npad_e = EPAD - E
    src3 = jnp.concatenate(
        [edge_index[0], jnp.zeros((npad_e,), jnp.int32)]).reshape(NW, ST, BT)
    dst3 = jnp.concatenate(
        [edge_index[1], jnp.full((npad_e,), N, jnp.int32)]).reshape(NW, ST, BT)
    xpad = jnp.pad(x, ((0, NPAD - N), (0, 0)))
    w1p = jnp.pad(W1, ((0, 0), (0, D - H)))
    w2p = jnp.pad(W2, ((0, D - H), (0, D - H)))
    b1p = jnp.pad(b1, (0, D - H)).reshape(1, D)
    b2p = jnp.pad(b2, (0, D - H)).reshape(1, D)
    wf1p = jnp.pad(Wf1, ((0, D - H), (0, 0)))
    bf1p = bf1.reshape(1, FC1)
    wf2p = jnp.pad(Wf2, ((0, 0), (0, 128 - NCLS)))
    bf2p = jnp.pad(bf2, (0, 128 - NCLS)).reshape(1, 128)

    degp = _deg_sc(dst3.reshape(NW, EW))
    deg = degp[0] + degp[1] + 1.0
    dinv2 = lax.rsqrt(deg)[:, None]

    u1 = _u1_tc(xpad, w1p, dinv2)
    s1 = _scatter_sc(u1, src3, dst3)
    u2 = _layer_tc(s1[0], s1[1], u1, dinv2, b1p, w2p)
    s2 = _scatter_sc(u2, src3, dst3)
    outp = _head_tc(s2[0], s2[1], u2, dinv2, b2p, wf1p, bf1p, wf2p, bf2p)
    return outp[0, :NCLS]


# speedup vs baseline: 14.6460x; 1.0071x over previous
import jax
import jax.numpy as jnp
from jax import lax
from jax.experimental import pallas as pl
from jax.experimental.pallas import tpu as pltpu
from jax.experimental.pallas import tpu_sc as plsc

N = 10000
NPAD = 10240
E = 320000
DF = 128
H = 50
D = 128
FC1 = 512
NCLS = 10

NC = 2
NS = 16
NW = NC * NS
BT = 128
ST = 79
EW = ST * BT
EPAD = NW * EW
RPT = NPAD // NS
STH = 40

_MESH = plsc.VectorSubcoreMesh(core_axis_name="c", subcore_axis_name="s",
                               num_cores=NC, num_subcores=NS)
_Z16 = 16


def _deg_kernel(dst_hbm, out_hbm, acc_sp, dstv, onesb, zb):
    c = lax.axis_index("c")
    s = lax.axis_index("s")
    w = c * NS + s

    for i in range(BT // 16):
        onesb[pl.ds(i * 16, 16)] = jnp.full((16,), 1.0, jnp.float32)

    def zero_z(i, carry):
        zb[pl.ds(i * 16, 16)] = jnp.zeros((16,), jnp.float32)
        return carry

    lax.fori_loop(0, RPT // 16, zero_z, 0)
    pltpu.sync_copy(zb, acc_sp.at[pl.ds(s * RPT, RPT)])

    pltpu.sync_copy(dst_hbm.at[w], dstv)
    plsc.subcore_barrier()

    def body(j, carry):
        pltpu.sync_copy(onesb, acc_sp.at[dstv.at[pl.ds(j * BT, BT)]],
                        add=True)
        return carry

    lax.fori_loop(0, ST, body, 0)
    plsc.subcore_barrier()

    pltpu.sync_copy(acc_sp.at[pl.ds(s * RPT, RPT)],
                    out_hbm.at[c, pl.ds(s * RPT, RPT)])


def _deg_sc(dst2):
    k = pl.kernel(
        _deg_kernel,
        out_type=jax.ShapeDtypeStruct((NC, NPAD), jnp.float32),
        mesh=_MESH,
        scratch_types=[
            pltpu.VMEM_SHARED((NPAD,), jnp.float32),
            pltpu.VMEM((EW,), jnp.int32),
            pltpu.VMEM((BT,), jnp.float32),
            pltpu.VMEM((RPT,), jnp.float32),
        ],
    )
    return k(dst2)


def _scatter_kernel(u_hbm, src_hbm, dst_hbm, out_hbm, acc_sp, srcv, dstv,
                    r0, r1, z16, g0, g1, s0, s1, zsem):
    c = lax.axis_index("c")
    s = lax.axis_index("s")
    w = c * NS + s

    for i in range(_Z16):
        for kk in range(D // 16):
            z16[i, pl.ds(kk * 16, 16)] = jnp.zeros((16,), jnp.float32)

    def zero_acc(j, carry):
        pltpu.async_copy(z16, acc_sp.at[pl.ds(s * RPT + j * _Z16, _Z16)],
                         zsem)
        return carry

    lax.fori_loop(0, RPT // _Z16, zero_acc, 0)

    pltpu.make_async_copy(u_hbm.at[pl.ds(0, RPT)],
                          acc_sp.at[pl.ds(s * RPT, RPT)], zsem).wait()
    plsc.subcore_barrier()

    def gather(j, rb, sem):
        pltpu.async_copy(u_hbm.at[srcv.at[j]], rb, sem)

    def scatter(j, rb, sem):
        pltpu.async_copy(rb, acc_sp.at[dstv.at[j]], sem, add=True)

    def wait_rows(sem):
        pltpu.make_async_copy(u_hbm.at[pl.ds(0, BT)], r0, sem).wait()

    def run_phase(base, count):
        pltpu.sync_copy(src_hbm.at[w, pl.ds(base, count)],
                        srcv.at[pl.ds(0, count)])
        pltpu.sync_copy(dst_hbm.at[w, pl.ds(base, count)],
                        dstv.at[pl.ds(0, count)])
        gather(0, r0, g0)
        gather(1, r1, g1)
        odd = count % 2
        iters = (count - 2 - odd) // 2

        def body(jj, carry):
            j = jj * 2
            wait_rows(g0)
            scatter(j, r0, s0)
            wait_rows(g1)
            scatter(j + 1, r1, s1)
            wait_rows(s0)
            gather(j + 2, r0, g0)
            wait_rows(s1)
            gather(j + 3, r1, g1)
            return carry

        lax.fori_loop(0, iters, body, 0)
        wait_rows(g0)
        scatter(2 * iters, r0, s0)
        wait_rows(g1)
        scatter(2 * iters + 1, r1, s1)
        wait_rows(s0)
        if odd:
            gather(count - 1, r0, g0)
            wait_rows(g0)
            scatter(count - 1, r0, s0)
            wait_rows(s0)
        wait_rows(s1)

    run_phase(0, STH)
    run_phase(STH, ST - STH)

    plsc.subcore_barrier()

    pltpu.sync_copy(acc_sp.at[pl.ds(s * RPT, RPT)],
                    out_hbm.at[c, pl.ds(s * RPT, RPT)])


def _scatter_sc(u, src3, dst3):
    k = pl.kernel(
        _scatter_kernel,
        out_type=jax.ShapeDtypeStruct((NC, NPAD, D), jnp.float32),
        mesh=_MESH,
        scratch_types=[
            pltpu.VMEM_SHARED((NPAD, D), jnp.float32),
            pltpu.VMEM((STH, BT), jnp.int32),
            pltpu.VMEM((STH, BT), jnp.int32),
            pltpu.VMEM((BT, D), jnp.float32),
            pltpu.VMEM((BT, D), jnp.float32),
            pltpu.VMEM((_Z16, D), jnp.float32),
            pltpu.SemaphoreType.DMA,
            pltpu.SemaphoreType.DMA,
            pltpu.SemaphoreType.DMA,
            pltpu.SemaphoreType.DMA,
            pltpu.SemaphoreType.DMA,
        ],
    )
    return k(u, src3, dst3)


def _u1_body(x_ref, w_ref, dinv_ref, o_ref):
    xw = jnp.dot(x_ref[...], w_ref[...], preferred_element_type=jnp.float32)
    o_ref[...] = dinv_ref[...] * xw


def _u1_tc(xpad, w1p, dinv2):
    grid = NPAD // 1024
    return pl.pallas_call(
        _u1_body,
        grid=(grid,),
        in_specs=[
            pl.BlockSpec((1024, DF), lambda i: (i, 0)),
            pl.BlockSpec((DF, D), lambda i: (0, 0)),
            pl.BlockSpec((1024, 1), lambda i: (i, 0)),
        ],
        out_specs=pl.BlockSpec((1024, D), lambda i: (i, 0)),
        out_shape=jax.ShapeDtypeStruct((NPAD, D), jnp.float32),
    )(xpad, w1p, dinv2)


def _elu(v):
    return jnp.where(v > 0, v, jnp.exp(jnp.minimum(v, 0.0)) - 1.0)


def _layer_body(sa_ref, sb_ref, u_ref, dinv_ref, b_ref, w_ref, o_ref):
    dinv = dinv_ref[...]
    p = dinv * (sa_ref[...] + sb_ref[...] + u_ref[...]) + b_ref[...]
    h = _elu(p)
    o_ref[...] = dinv * jnp.dot(h, w_ref[...],
                                preferred_element_type=jnp.float32)


def _layer_tc(sa, sb, u1, dinv2, b1p, w2p):
    grid = NPAD // 1024
    return pl.pallas_call(
        _layer_body,
        grid=(grid,),
        in_specs=[
            pl.BlockSpec((1024, D), lambda i: (i, 0)),
            pl.BlockSpec((1024, D), lambda i: (i, 0)),
            pl.BlockSpec((1024, D), lambda i: (i, 0)),
            pl.BlockSpec((1024, 1), lambda i: (i, 0)),
            pl.BlockSpec((1, D), lambda i: (0, 0)),
            pl.BlockSpec((D, D), lambda i: (0, 0)),
        ],
        out_specs=pl.BlockSpec((1024, D), lambda i: (i, 0)),
        out_shape=jax.ShapeDtypeStruct((NPAD, D), jnp.float32),
    )(sa, sb, u1, dinv2, b1p, w2p)


def _head_body(sa_ref, sb_ref, u_ref, dinv_ref, b_ref, wf1_ref, bf1_ref,
               wf2_ref, bf2_ref, o_ref, g_acc):
    i = pl.program_id(0)

    @pl.when(i == 0)
    def _():
        g_acc[...] = jnp.zeros_like(g_acc)

    dinv = dinv_ref[...]
    p = dinv * (sa_ref[...] + sb_ref[...] + u_ref[...]) + b_ref[...]
    h = _elu(p)
    rows = i * 1024 + jax.lax.broadcasted_iota(jnp.int32, (1024, D), 0)
    h = jnp.where(rows < N, h, 0.0)
    g_acc[...] += jnp.sum(h, axis=0, keepdims=True)

    @pl.when(i == pl.num_programs(0) - 1)
    def _():
        g = g_acc[...]
        t = jnp.maximum(
            jnp.dot(g, wf1_ref[...], preferred_element_type=jnp.float32)
            + bf1_ref[...], 0.0)
        logits = jnp.dot(t, wf2_ref[...],
                         preferred_element_type=jnp.float32) + bf2_ref[...]
        cols = jax.lax.broadcasted_iota(jnp.int32, (1, 128), 1)
        logits = jnp.where(cols < NCLS, logits, -1e30)
        m = jnp.max(logits, axis=1, keepdims=True)
        e = jnp.exp(logits - m)
        o_ref[...] = e / jnp.sum(e, axis=1, keepdims=True)


def _head_tc(sa, sb, u2, dinv2, b2p, wf1p, bf1p, wf2p, bf2p):
    grid = NPAD // 1024
    return pl.pallas_call(
        _head_body,
        grid=(grid,),
        in_specs=[
            pl.BlockSpec((1024, D), lambda i: (i, 0)),
            pl.BlockSpec((1024, D), lambda i: (i, 0)),
            pl.BlockSpec((1024, D), lambda i: (i, 0)),
            pl.BlockSpec((1024, 1), lambda i: (i, 0)),
            pl.BlockSpec((1, D), lambda i: (0, 0)),
            pl.BlockSpec((D, FC1), lambda i: (0, 0)),
            pl.BlockSpec((1, FC1), lambda i: (0, 0)),
            pl.BlockSpec((FC1, 128), lambda i: (0, 0)),
            pl.BlockSpec((1, 128), lambda i: (0, 0)),
        ],
        out_specs=pl.BlockSpec((1, 128), lambda i: (0, 0)),
        out_shape=jax.ShapeDtypeStruct((1, 128), jnp.float32),
        scratch_shapes=[pltpu.VMEM((1, D), jnp.float32)],
    )(sa, sb, u2, dinv2, b2p, wf1p, bf1p, wf2p, bf2p)


def kernel(x, edge_index, W1, b1, W2, b2, Wf1, bf1, Wf2, bf2):
    npad_e = EPAD - E
    src3 = jnp.concatenate(
        [edge_index[0], jnp.zeros((npad_e,), jnp.int32)]).reshape(NW, ST, BT)
    dst3 = jnp.concatenate(
        [edge_index[1], jnp.full((npad_e,), N, jnp.int32)]).reshape(NW, ST, BT)
    xpad = jnp.pad(x, ((0, NPAD - N), (0, 0)))
    w1p = jnp.pad(W1, ((0, 0), (0, D - H)))
    w2p = jnp.pad(W2, ((0, D - H), (0, D - H)))
    b1p = jnp.pad(b1, (0, D - H)).reshape(1, D)
    b2p = jnp.pad(b2, (0, D - H)).reshape(1, D)
    wf1p = jnp.pad(Wf1, ((0, D - H), (0, 0)))
    bf1p = bf1.reshape(1, FC1)
    wf2p = jnp.pad(Wf2, ((0, 0), (0, 128 - NCLS)))
    bf2p = jnp.pad(bf2, (0, 128 - NCLS)).reshape(1, 128)

    degp = _deg_sc(dst3.reshape(NW, EW))
    deg = degp[0] + degp[1] + 1.0
    dinv2 = lax.rsqrt(deg)[:, None]

    u1 = _u1_tc(xpad, w1p, dinv2)
    s1 = _scatter_sc(u1, src3, dst3)
    u2 = _layer_tc(s1[0], s1[1], u1, dinv2, b1p, w2p)
    s2 = _scatter_sc(u2, src3, dst3)
    outp = _head_tc(s2[0], s2[1], u2, dinv2, b2p, wf1p, bf1p, wf2p, bf2p)
    return outp[0, :NCLS]
